# baseline (device time: 53883 ns/iter reference)
import jax
import jax.numpy as jnp
from jax import lax
from jax.experimental import pallas as pl
from jax.experimental.pallas import tpu as pltpu

N_Z = 4
N_XY = 8
CHUNK = 1024
D = 1024
SL = D // N_XY
HR = CHUNK // 2


def kernel(partial, gamma):
    gamma2 = gamma.reshape(1, D)

    def body(x_ref, g_ref, out_ref, ph1_ref, y_ref,
             p1_send, p1_recv, cw_send, cw_recv, ccw_send, ccw_recv,
             xy_bar):
        my_x = lax.axis_index("x")
        my_y = lax.axis_index("y")
        my_z = lax.axis_index("z")
        zl = (my_z + N_Z - 1) % N_Z
        zr = (my_z + 1) % N_Z

        p = jnp.where(my_x == 0, my_y, 7 - my_y)

        def ring_coords(q):
            q = q % N_XY
            return q // 4, jnp.where(q < 4, q, 7 - q)

        cw_x, cw_y = ring_coords(p + 1)
        ccw_x, ccw_y = ring_coords(p + N_XY - 1)

        barrier = pltpu.get_barrier_semaphore()
        for nz in (zl, zr):
            pl.semaphore_signal(
                barrier, inc=1,
                device_id=(my_x, my_y, nz),
                device_id_type=pl.DeviceIdType.MESH,
            )
        pl.semaphore_wait(barrier, 2)

        for qx, qy in ((cw_x, cw_y), (ccw_x, ccw_y)):
            pl.semaphore_signal(
                xy_bar, inc=1,
                device_id=(qx, qy, my_z),
                device_id_type=pl.DeviceIdType.MESH,
            )

        col0 = p * SL

        ph1_rd = [[None, None] for _ in range(N_Z - 1)]

        def start_ph1(s, t):
            if s == 0:
                src = x_ref.at[0, pl.ds(zl * CHUNK + t * HR, HR), pl.ds(col0, SL)]
            else:
                src = ph1_ref.at[s - 1, pl.ds(t * HR, HR), :]
            rdma = pltpu.make_async_remote_copy(
                src_ref=src,
                dst_ref=ph1_ref.at[s, pl.ds(t * HR, HR), :],
                send_sem=p1_send.at[t, s],
                recv_sem=p1_recv.at[t, s],
                device_id=(my_x, my_y, zr),
                device_id_type=pl.DeviceIdType.MESH,
            )
            rdma.start()
            ph1_rd[s][t] = rdma

        start_ph1(0, 0)
        start_ph1(0, 1)
        for s in range(N_Z - 2):
            c = (my_z + 2 * N_Z - s - 2) % N_Z
            for t in (0, 1):
                ph1_rd[s][t].wait_recv()
                local = x_ref[0, pl.ds(c * CHUNK + t * HR, HR), pl.ds(col0, SL)]
                ph1_ref[s, pl.ds(t * HR, HR), :] = (
                    ph1_ref[s, pl.ds(t * HR, HR), :] + local
                )
                start_ph1(s + 1, t)

        pl.semaphore_wait(xy_bar, 2)

        def slice_ref(q, t):
            return y_ref.at[pl.ds(t * HR, HR), pl.ds((q % N_XY) * SL, SL)]

        def ssq_of(q):
            v = y_ref[:, pl.ds((q % N_XY) * SL, SL)]
            return jnp.sum(v * v, axis=-1, keepdims=True)

        streams = [
            ("cw", 0, 4, cw_send, cw_recv, (cw_x, cw_y)),
            ("cw", 1, 3, cw_send, cw_recv, (cw_x, cw_y)),
            ("ccw", 0, 3, ccw_send, ccw_recv, (ccw_x, ccw_y)),
            ("ccw", 1, 4, ccw_send, ccw_recv, (ccw_x, ccw_y)),
        ]
        ph2_rd = {}

        def start_ph2(si, h):
            d, t, _, ssem, rsem, (qx, qy) = streams[si]
            q = (p - h) if d == "cw" else (p + h)
            rdma = pltpu.make_async_remote_copy(
                src_ref=slice_ref(q, t),
                dst_ref=slice_ref(q, t),
                send_sem=ssem.at[t, h],
                recv_sem=rsem.at[t, h],
                device_id=(qx, qy, my_z),
                device_id_type=pl.DeviceIdType.MESH,
            )
            rdma.start()
            ph2_rd[(si, h)] = rdma

        s_last = N_Z - 2
        c = my_z
        for t, starts in ((0, (0, 2)), (1, (1, 3))):
            ph1_rd[s_last][t].wait_recv()
            local = x_ref[0, pl.ds(c * CHUNK + t * HR, HR), pl.ds(col0, SL)]
            y_ref[pl.ds(t * HR, HR), pl.ds(col0, SL)] = (
                ph1_ref[s_last, pl.ds(t * HR, HR), :] + local
            )
            for si in starts:
                start_ph2(si, 0)

        def ssq_half(q, t):
            v = y_ref[pl.ds(t * HR, HR), pl.ds((q % N_XY) * SL, SL)]
            return jnp.sum(v * v, axis=-1, keepdims=True)

        ssq = None
        for h in range(3):
            for si in range(4):
                nh = streams[si][2]
                if h < nh:
                    ph2_rd[(si, h)].wait_recv()
                    if h + 1 < nh:
                        start_ph2(si, h + 1)
            if h == 0:
                ssq = ssq_of(p)
            else:
                ssq = ssq + ssq_of(p - h) + ssq_of(p + h)
        ssq = ssq + ssq_of(p - 3) + ssq_of(p + 3)

        inv_d = 1.0 / D
        ph2_rd[(0, 3)].wait_recv()
        ms_t = (ssq[:HR] + ssq_half(p + 4, 0)) * inv_d + 1e-6
        out_ref[pl.ds(0, HR), :] = (
            y_ref[pl.ds(0, HR), :] * lax.rsqrt(ms_t) * g_ref[0, :]
        )
        ph2_rd[(3, 3)].wait_recv()
        ms_b = (ssq[HR:] + ssq_half(p + 4, 1)) * inv_d + 1e-6
        out_ref[pl.ds(HR, HR), :] = (
            y_ref[pl.ds(HR, HR), :] * lax.rsqrt(ms_b) * g_ref[0, :]
        )

        for row in ph1_rd:
            for rdma in row:
                rdma.wait_send()
        for rdma in ph2_rd.values():
            rdma.wait_send()

    return pl.pallas_call(
        body,
        out_shape=jax.ShapeDtypeStruct((CHUNK, D), jnp.float32),
        in_specs=[
            pl.BlockSpec(memory_space=pltpu.VMEM),
            pl.BlockSpec(memory_space=pltpu.VMEM),
        ],
        out_specs=pl.BlockSpec(memory_space=pltpu.VMEM),
        scratch_shapes=[
            pltpu.VMEM((N_Z - 1, CHUNK, SL), jnp.float32),
            pltpu.VMEM((CHUNK, D), jnp.float32),
            pltpu.SemaphoreType.DMA((2, N_Z - 1)),
            pltpu.SemaphoreType.DMA((2, N_Z - 1)),
            pltpu.SemaphoreType.DMA((2, 4)),
            pltpu.SemaphoreType.DMA((2, 4)),
            pltpu.SemaphoreType.DMA((2, 4)),
            pltpu.SemaphoreType.DMA((2, 4)),
            pltpu.SemaphoreType.REGULAR,
        ],
        compiler_params=pltpu.CompilerParams(collective_id=0),
    )(partial, gamma2)


# device time: 52744 ns/iter; 1.0216x vs baseline; 1.0216x over previous
import jax
import jax.numpy as jnp
from jax import lax
from jax.experimental import pallas as pl
from jax.experimental.pallas import tpu as pltpu

N_Z = 4
N_XY = 8
CHUNK = 1024
D = 1024
SL = D // N_XY
HR = CHUNK // 2


def kernel(partial, gamma):
    gamma2 = gamma.reshape(1, D)

    def body(x_ref, g_ref, out_ref, ph1_ref, y_ref,
             p1_send, p1_recv, cw_send, cw_recv, ccw_send, ccw_recv,
             xy_bar):
        my_x = lax.axis_index("x")
        my_y = lax.axis_index("y")
        my_z = lax.axis_index("z")
        zl = (my_z + N_Z - 1) % N_Z
        zr = (my_z + 1) % N_Z

        p = jnp.where(my_x == 0, my_y, 7 - my_y)

        def ring_coords(q):
            q = q % N_XY
            return q // 4, jnp.where(q < 4, q, 7 - q)

        cw_x, cw_y = ring_coords(p + 1)
        ccw_x, ccw_y = ring_coords(p + N_XY - 1)

        barrier = pltpu.get_barrier_semaphore()
        for nz in (zl, zr):
            pl.semaphore_signal(
                barrier, inc=1,
                device_id=(my_x, my_y, nz),
                device_id_type=pl.DeviceIdType.MESH,
            )
        pl.semaphore_wait(barrier, 2)

        for qx, qy in ((cw_x, cw_y), (ccw_x, ccw_y)):
            pl.semaphore_signal(
                xy_bar, inc=1,
                device_id=(qx, qy, my_z),
                device_id_type=pl.DeviceIdType.MESH,
            )

        col0 = p * SL

        ph1_rd = [[None, None] for _ in range(N_Z - 1)]

        def start_ph1(s, t):
            if s == 0:
                src = x_ref.at[0, pl.ds(zl * CHUNK + t * HR, HR), pl.ds(col0, SL)]
            else:
                src = ph1_ref.at[s - 1, pl.ds(t * HR, HR), :]
            rdma = pltpu.make_async_remote_copy(
                src_ref=src,
                dst_ref=ph1_ref.at[s, pl.ds(t * HR, HR), :],
                send_sem=p1_send.at[t, s],
                recv_sem=p1_recv.at[t, s],
                device_id=(my_x, my_y, zr),
                device_id_type=pl.DeviceIdType.MESH,
            )
            rdma.start()
            ph1_rd[s][t] = rdma

        start_ph1(0, 0)
        start_ph1(0, 1)
        for s in range(N_Z - 2):
            c = (my_z + 2 * N_Z - s - 2) % N_Z
            for t in (0, 1):
                ph1_rd[s][t].wait_recv()
                local = x_ref[0, pl.ds(c * CHUNK + t * HR, HR), pl.ds(col0, SL)]
                ph1_ref[s, pl.ds(t * HR, HR), :] = (
                    ph1_ref[s, pl.ds(t * HR, HR), :] + local
                )
                start_ph1(s + 1, t)

        pl.semaphore_wait(xy_bar, 2)

        def slice_ref(q, t):
            return y_ref.at[pl.ds(t * HR, HR), pl.ds((q % N_XY) * SL, SL)]

        def ssq_of(q):
            v = y_ref[:, pl.ds((q % N_XY) * SL, SL)]
            return jnp.sum(v * v, axis=-1, keepdims=True)

        streams = [
            ("cw", 0, 4, cw_send, cw_recv, (cw_x, cw_y)),
            ("ccw", 1, 4, ccw_send, ccw_recv, (ccw_x, ccw_y)),
            ("cw", 1, 3, cw_send, cw_recv, (cw_x, cw_y)),
            ("ccw", 0, 3, ccw_send, ccw_recv, (ccw_x, ccw_y)),
        ]
        ph2_rd = {}

        def start_ph2(si, h):
            d, t, _, ssem, rsem, (qx, qy) = streams[si]
            q = (p - h) if d == "cw" else (p + h)
            rdma = pltpu.make_async_remote_copy(
                src_ref=slice_ref(q, t),
                dst_ref=slice_ref(q, t),
                send_sem=ssem.at[t, h],
                recv_sem=rsem.at[t, h],
                device_id=(qx, qy, my_z),
                device_id_type=pl.DeviceIdType.MESH,
            )
            rdma.start()
            ph2_rd[(si, h)] = rdma

        s_last = N_Z - 2
        c = my_z
        for t, starts in ((0, (0, 3)), (1, (1, 2))):
            ph1_rd[s_last][t].wait_recv()
            local = x_ref[0, pl.ds(c * CHUNK + t * HR, HR), pl.ds(col0, SL)]
            y_ref[pl.ds(t * HR, HR), pl.ds(col0, SL)] = (
                ph1_ref[s_last, pl.ds(t * HR, HR), :] + local
            )
            for si in starts:
                start_ph2(si, 0)

        def ssq_half(q, t):
            v = y_ref[pl.ds(t * HR, HR), pl.ds((q % N_XY) * SL, SL)]
            return jnp.sum(v * v, axis=-1, keepdims=True)

        ssq = None
        for h in range(3):
            for si in range(4):
                nh = streams[si][2]
                if h < nh:
                    ph2_rd[(si, h)].wait_recv()
                    if h + 1 < nh:
                        start_ph2(si, h + 1)
            if h == 0:
                ssq = ssq_of(p)
            else:
                ssq = ssq + ssq_of(p - h) + ssq_of(p + h)
        ssq = ssq + ssq_of(p - 3) + ssq_of(p + 3)

        inv_d = 1.0 / D
        ph2_rd[(0, 3)].wait_recv()
        ms_t = (ssq[:HR] + ssq_half(p + 4, 0)) * inv_d + 1e-6
        out_ref[pl.ds(0, HR), :] = (
            y_ref[pl.ds(0, HR), :] * lax.rsqrt(ms_t) * g_ref[0, :]
        )
        ph2_rd[(1, 3)].wait_recv()
        ms_b = (ssq[HR:] + ssq_half(p + 4, 1)) * inv_d + 1e-6
        out_ref[pl.ds(HR, HR), :] = (
            y_ref[pl.ds(HR, HR), :] * lax.rsqrt(ms_b) * g_ref[0, :]
        )

        for row in ph1_rd:
            for rdma in row:
                rdma.wait_send()
        for rdma in ph2_rd.values():
            rdma.wait_send()

    return pl.pallas_call(
        body,
        out_shape=jax.ShapeDtypeStruct((CHUNK, D), jnp.float32),
        in_specs=[
            pl.BlockSpec(memory_space=pltpu.VMEM),
            pl.BlockSpec(memory_space=pltpu.VMEM),
        ],
        out_specs=pl.BlockSpec(memory_space=pltpu.VMEM),
        scratch_shapes=[
            pltpu.VMEM((N_Z - 1, CHUNK, SL), jnp.float32),
            pltpu.VMEM((CHUNK, D), jnp.float32),
            pltpu.SemaphoreType.DMA((2, N_Z - 1)),
            pltpu.SemaphoreType.DMA((2, N_Z - 1)),
            pltpu.SemaphoreType.DMA((2, 4)),
            pltpu.SemaphoreType.DMA((2, 4)),
            pltpu.SemaphoreType.DMA((2, 4)),
            pltpu.SemaphoreType.DMA((2, 4)),
            pltpu.SemaphoreType.REGULAR,
        ],
        compiler_params=pltpu.CompilerParams(collective_id=0),
    )(partial, gamma2)
